# baseline (speedup 1.0000x reference)
import functools

import jax
import jax.numpy as jnp
from jax import lax
from jax.experimental import pallas as pl
from jax.experimental.pallas import tpu as pltpu
from jax.experimental.pallas import tpu_sc as plsc

N_NODES = 10000
N_EDGES = 320000
FT = 128

NC = 2
NS = 16
NW = NC * NS

CHUNK = 128
E_PER_TILE = 10240
E_PAD = NW * E_PER_TILE
IDX_ROWS = E_PAD // CHUNK // NW
BLK_ROWS = 16
N_BLKS = IDX_ROWS // BLK_ROWS
PAIRS = IDX_ROWS // 2
ROWS_PER_TILE = 624
ROWS_REM = N_NODES - NS * ROWS_PER_TILE
VREGS_PER_ROW = FT // 16


def _sc_agg_body(nf_hbm, src_hbm, dst_hbm, w_hbm, out_hbm,
                 src_blk, dst_blk, w_blk, rows_a, rows_b,
                 acc, gsem_a, gsem_b, ssem_a, ssem_b, isem):
    c = lax.axis_index("c")
    s = lax.axis_index("s")
    wid = s * NC + c
    irow0 = wid * IDX_ROWS

    zero = jnp.zeros((16,), jnp.float32)

    def _zero_row(e, _):
        for r in range(VREGS_PER_ROW):
            rows_a[e, pl.ds(r * 16, 16)] = zero
        return 0

    lax.fori_loop(0, CHUNK, _zero_row, 0)
    row0 = s * ROWS_PER_TILE
    for j in range(ROWS_PER_TILE // CHUNK):
        pltpu.sync_copy(rows_a, acc.at[pl.ds(row0 + j * CHUNK, CHUNK)])
    _rem = ROWS_PER_TILE % CHUNK
    if _rem:
        pltpu.sync_copy(rows_a.at[pl.ds(0, _rem)],
                        acc.at[pl.ds(row0 + ROWS_PER_TILE - _rem, _rem)])

    @pl.when(s == 0)
    def _zero_tail():
        pltpu.sync_copy(rows_a.at[pl.ds(0, ROWS_REM)],
                        acc.at[pl.ds(NS * ROWS_PER_TILE, ROWS_REM)])

    plsc.subcore_barrier()

    def _fire_idx_block(nb):
        buf = nb % 2
        r = irow0 + nb * BLK_ROWS
        pltpu.async_copy(src_hbm.at[pl.ds(r, BLK_ROWS)], src_blk.at[buf], isem)
        pltpu.async_copy(dst_hbm.at[pl.ds(r, BLK_ROWS)], dst_blk.at[buf], isem)
        pltpu.async_copy(w_hbm.at[pl.ds(r, BLK_ROWS)], w_blk.at[buf], isem)

    def _wait_idx_block(nb):
        buf = nb % 2
        r = irow0 + nb * BLK_ROWS
        pltpu.make_async_copy(src_hbm.at[pl.ds(r, BLK_ROWS)], src_blk.at[buf], isem).wait()
        pltpu.make_async_copy(dst_hbm.at[pl.ds(r, BLK_ROWS)], dst_blk.at[buf], isem).wait()
        pltpu.make_async_copy(w_hbm.at[pl.ds(r, BLK_ROWS)], w_blk.at[buf], isem).wait()

    def _scale(rows_v, buf, row):
        def _scale_group(g, _):
            wv = w_blk[buf, row, pl.ds(g * 16, 16)]
            for i in range(16):
                e = g * 16 + i
                w = wv[i]
                for r in range(VREGS_PER_ROW):
                    sl = pl.ds(r * 16, 16)
                    rows_v[e, sl] = rows_v[e, sl] * w
            return 0

        lax.fori_loop(0, CHUNK // 16, _scale_group, 0)

    _fire_idx_block(0)
    _wait_idx_block(0)
    pltpu.async_copy(nf_hbm.at[src_blk.at[0, 0]], rows_a, gsem_a)
    pltpu.async_copy(nf_hbm.at[src_blk.at[0, 1]], rows_b, gsem_b)

    def _pair(p, _):
        c0 = 2 * p
        jb = c0 // BLK_ROWS
        buf = jb % 2
        r0 = c0 % BLK_ROWS
        r1 = r0 + 1

        @pl.when(jnp.logical_and(r0 == 0, jb + 1 < N_BLKS))
        def _stage_next():
            _fire_idx_block(jb + 1)

        pltpu.make_async_copy(nf_hbm.at[src_blk.at[buf, r0]], rows_a, gsem_a).wait()
        _scale(rows_a, buf, r0)
        pltpu.async_copy(rows_a, acc.at[dst_blk.at[buf, r0]], ssem_a, add=True)
        pltpu.make_async_copy(nf_hbm.at[src_blk.at[buf, r1]], rows_b, gsem_b).wait()
        _scale(rows_b, buf, r1)
        pltpu.async_copy(rows_b, acc.at[dst_blk.at[buf, r1]], ssem_b, add=True)

        @pl.when(p + 1 < PAIRS)
        def _prefetch():
            c2 = c0 + 2
            jb2 = c2 // BLK_ROWS
            buf2 = jb2 % 2
            r2 = c2 % BLK_ROWS

            @pl.when(r0 == BLK_ROWS - 2)
            def _await_next_idx():
                _wait_idx_block(jb + 1)

            pltpu.make_async_copy(rows_a, acc.at[dst_blk.at[buf, r0]], ssem_a).wait()
            pltpu.async_copy(nf_hbm.at[src_blk.at[buf2, r2]], rows_a, gsem_a)
            pltpu.make_async_copy(rows_b, acc.at[dst_blk.at[buf, r1]], ssem_b).wait()
            pltpu.async_copy(nf_hbm.at[src_blk.at[buf2, r2 + 1]], rows_b, gsem_b)

        return 0

    lax.fori_loop(0, PAIRS, _pair, 0)
    pltpu.make_async_copy(rows_a, acc.at[dst_blk.at[0, 0]], ssem_a).wait()
    pltpu.make_async_copy(rows_b, acc.at[dst_blk.at[0, 1]], ssem_b).wait()
    plsc.subcore_barrier()

    pltpu.sync_copy(acc.at[pl.ds(row0, ROWS_PER_TILE)],
                    out_hbm.at[c, pl.ds(row0, ROWS_PER_TILE)])

    @pl.when(s == 0)
    def _write_tail():
        pltpu.sync_copy(acc.at[pl.ds(NS * ROWS_PER_TILE, ROWS_REM)],
                        out_hbm.at[c, pl.ds(NS * ROWS_PER_TILE, ROWS_REM)])


@functools.partial(
    pl.kernel,
    out_type=jax.ShapeDtypeStruct((NC, N_NODES, FT), jnp.float32),
    mesh=plsc.VectorSubcoreMesh(core_axis_name="c", subcore_axis_name="s"),
    scratch_types=[
        pltpu.VMEM((2, BLK_ROWS, CHUNK), jnp.int32),
        pltpu.VMEM((2, BLK_ROWS, CHUNK), jnp.int32),
        pltpu.VMEM((2, BLK_ROWS, CHUNK), jnp.float32),
        pltpu.VMEM((CHUNK, FT), jnp.float32),
        pltpu.VMEM((CHUNK, FT), jnp.float32),
        pltpu.VMEM_SHARED((N_NODES, FT), jnp.float32),
        pltpu.SemaphoreType.DMA,
        pltpu.SemaphoreType.DMA,
        pltpu.SemaphoreType.DMA,
        pltpu.SemaphoreType.DMA,
        pltpu.SemaphoreType.DMA,
    ],
)
def _sc_agg(*args):
    _sc_agg_body(*args)


ROW_BLK = 1000


def _tc_post_body(p_ref, w_ref, b_ref, o_ref):
    p = p_ref[0] + p_ref[1]
    y = lax.dot_general(p, w_ref[...], (((1,), (1,)), ((), ())),
                        preferred_element_type=jnp.float32)
    o_ref[...] = jnp.maximum(y + b_ref[...], 0.0)


def _tc_post(partials, W, b2d):
    return pl.pallas_call(
        _tc_post_body,
        out_shape=jax.ShapeDtypeStruct((N_NODES, FT), jnp.float32),
        grid=(N_NODES // ROW_BLK,),
        in_specs=[
            pl.BlockSpec((NC, ROW_BLK, FT), lambda i: (0, i, 0)),
            pl.BlockSpec((FT, FT), lambda i: (0, 0)),
            pl.BlockSpec((1, FT), lambda i: (0, 0)),
        ],
        out_specs=pl.BlockSpec((ROW_BLK, FT), lambda i: (i, 0)),
    )(partials, W, b2d)


def kernel(node_fts, edge_index, edge_weight, W, b):
    src = edge_index[1].astype(jnp.int32)
    dst = edge_index[0].astype(jnp.int32)
    npad = E_PAD - N_EDGES
    zi = jnp.zeros((npad,), jnp.int32)
    src_p = jnp.concatenate([src, zi]).reshape(E_PAD // CHUNK, CHUNK)
    dst_p = jnp.concatenate([dst, zi]).reshape(E_PAD // CHUNK, CHUNK)
    w_p = jnp.concatenate([edge_weight, jnp.zeros((npad,), jnp.float32)])
    w_p = w_p.reshape(E_PAD // CHUNK, CHUNK)
    partials = _sc_agg(node_fts, src_p, dst_p, w_p)
    return _tc_post(partials, W, b.reshape(1, FT))
